# baseline (speedup 1.0000x reference)
import jax
import jax.numpy as jnp
from jax.experimental import pallas as pl
from jax.experimental.pallas import tpu as pltpu


def _dma_copy(x_ref, o_ref, sem):
    cp = pltpu.make_async_copy(x_ref, o_ref, sem)
    cp.start()
    cp.wait()


def kernel(x, edge_index, train):
    del edge_index, train
    return pl.pallas_call(
        _dma_copy,
        in_specs=[pl.BlockSpec(memory_space=pl.ANY)],
        out_specs=pl.BlockSpec(memory_space=pl.ANY),
        out_shape=jax.ShapeDtypeStruct(x.shape, x.dtype),
        scratch_shapes=[pltpu.SemaphoreType.DMA],
    )(x)


# speedup vs baseline: 18.6742x; 18.6742x over previous
import jax
import jax.numpy as jnp
from jax.experimental import pallas as pl
from jax.experimental.pallas import tpu as pltpu


def _copy_block(x_ref, o_ref):
    o_ref[...] = x_ref[...]


def kernel(x, edge_index, train):
    del edge_index, train
    n, d = x.shape
    block = 1000
    return pl.pallas_call(
        _copy_block,
        grid=(n // block,),
        in_specs=[pl.BlockSpec((block, d), lambda i: (i, 0))],
        out_specs=pl.BlockSpec((block, d), lambda i: (i, 0)),
        out_shape=jax.ShapeDtypeStruct((n, d), x.dtype),
        compiler_params=pltpu.CompilerParams(
            dimension_semantics=("parallel",),
        ),
    )(x)


# speedup vs baseline: 36.3665x; 1.9474x over previous
import jax
import jax.numpy as jnp
from jax.experimental import pallas as pl
from jax.experimental.pallas import tpu as pltpu

_NCHUNK = 16


def _dma_chase(x_ref, o_ref, buf, in_sems, out_sems):
    n = x_ref.shape[0]
    rows = n // _NCHUNK

    def in_cp(i):
        sl = pl.ds(i * rows, rows)
        return pltpu.make_async_copy(x_ref.at[sl], buf.at[sl], in_sems.at[i])

    def out_cp(i):
        sl = pl.ds(i * rows, rows)
        return pltpu.make_async_copy(buf.at[sl], o_ref.at[sl], out_sems.at[i])

    for i in range(_NCHUNK):
        in_cp(i).start()
    for i in range(_NCHUNK):
        in_cp(i).wait()
        out_cp(i).start()
    for i in range(_NCHUNK):
        out_cp(i).wait()


def kernel(x, edge_index, train):
    del edge_index, train
    n, d = x.shape
    return pl.pallas_call(
        _dma_chase,
        in_specs=[pl.BlockSpec(memory_space=pl.ANY)],
        out_specs=pl.BlockSpec(memory_space=pl.ANY),
        out_shape=jax.ShapeDtypeStruct((n, d), x.dtype),
        scratch_shapes=[
            pltpu.VMEM((n, d), x.dtype),
            pltpu.SemaphoreType.DMA((_NCHUNK,)),
            pltpu.SemaphoreType.DMA((_NCHUNK,)),
        ],
    )(x)
